# baseline (speedup 1.0000x reference)
import jax
import jax.numpy as jnp
from jax.experimental import pallas as pl
from jax.experimental.pallas import tpu as pltpu

_B = 128
_V = 100000
_C = 8192
_NCHUNK = (_V + _C - 1) // _C


def _lse_body(a_ref, x_ref, o_ref, m_ref, s_ref, g_ref):
    j = pl.program_id(0)

    @pl.when(j == 0)
    def _init():
        m_ref[...] = jnp.full((_B, 1), -jnp.inf, jnp.float32)
        s_ref[...] = jnp.zeros((_B, 1), jnp.float32)
        g_ref[...] = jnp.zeros((_B, 1), jnp.float32)

    x = x_ref[...]
    col = j * _C + jax.lax.broadcasted_iota(jnp.int32, (_B, _C), 1)
    xm = jnp.where(col < _V, x, -jnp.inf)
    m_old = m_ref[...]
    m_new = jnp.maximum(m_old, jnp.max(xm, axis=-1, keepdims=True))
    s_ref[...] = s_ref[...] * jnp.exp(m_old - m_new) + jnp.sum(
        jnp.exp(xm - m_new), axis=-1, keepdims=True)
    m_ref[...] = m_new
    a = a_ref[...]
    g_ref[...] += jnp.sum(jnp.where(col == a, x, 0.0), axis=-1, keepdims=True)

    @pl.when(j == _NCHUNK - 1)
    def _fin():
        o_ref[...] = g_ref[...] - (m_ref[...] + jnp.log(s_ref[...]))


def kernel(logits, actions):
    a = actions.astype(jnp.int32)
    return pl.pallas_call(
        _lse_body,
        grid=(_NCHUNK,),
        in_specs=[
            pl.BlockSpec((_B, 1), lambda j: (0, 0)),
            pl.BlockSpec((_B, _C), lambda j: (0, j)),
        ],
        out_specs=pl.BlockSpec((_B, 1), lambda j: (0, 0)),
        out_shape=jax.ShapeDtypeStruct((_B, 1), jnp.float32),
        scratch_shapes=[pltpu.VMEM((_B, 1), jnp.float32)] * 3,
    )(a, logits)
